# baseline (speedup 1.0000x reference)
import jax
import jax.numpy as jnp
from jax import lax
from jax.experimental import pallas as pl
from jax.experimental.pallas import tpu as pltpu
from jax.experimental.pallas import tpu_sc as plsc

D_MODEL = 768
NUM_INDICES = 4 * 8192

_info = plsc.get_sparse_core_info()
_NC, _NS = _info.num_cores, _info.num_subcores
_NW = _NC * _NS
_PER_W = NUM_INDICES // _NW
_CHUNK = 64
_NCHUNK = _PER_W // _CHUNK


def _gather_body(table_hbm, idx_hbm, out_hbm, idx_v, rows_a, rows_b, sem_idx,
                 sem_ga, sem_gb, sem_oa, sem_ob):
    wid = lax.axis_index("s") * _NC + lax.axis_index("c")
    base = wid * _PER_W

    pltpu.async_copy(idx_hbm.at[wid], idx_v, sem_idx).wait()

    bufs = (rows_a, rows_b)
    gsems = (sem_ga, sem_gb)
    osems = (sem_oa, sem_ob)

    def gather(j):
        return pltpu.make_async_copy(
            table_hbm.at[idx_v.at[j]], bufs[j % 2], gsems[j % 2])

    def scatter(j):
        return pltpu.make_async_copy(
            bufs[j % 2], out_hbm.at[pl.ds(base + j * _CHUNK, _CHUNK)],
            osems[j % 2])

    gather(0).start()
    for j in range(_NCHUNK):
        gather(j).wait()
        scatter(j).start()
        if j + 1 < _NCHUNK:
            if j >= 1:
                scatter(j - 1).wait()
            gather(j + 1).start()
    scatter(_NCHUNK - 2).wait()
    scatter(_NCHUNK - 1).wait()


def kernel(indices, emb_weight):
    b, l = indices.shape
    idx_flat = indices.reshape(_NW, _NCHUNK, _CHUNK).astype(jnp.int32)

    mesh = plsc.VectorSubcoreMesh(core_axis_name="c", subcore_axis_name="s")
    run = pl.kernel(
        _gather_body,
        mesh=mesh,
        out_type=jax.ShapeDtypeStruct((NUM_INDICES, D_MODEL), jnp.float32),
        scratch_types=[
            pltpu.VMEM((_NCHUNK, _CHUNK), jnp.int32),
            pltpu.VMEM((_CHUNK, D_MODEL), jnp.float32),
            pltpu.VMEM((_CHUNK, D_MODEL), jnp.float32),
            pltpu.SemaphoreType.DMA,
            pltpu.SemaphoreType.DMA,
            pltpu.SemaphoreType.DMA,
            pltpu.SemaphoreType.DMA,
            pltpu.SemaphoreType.DMA,
        ],
    )
    out = run(emb_weight, idx_flat)
    return out.reshape(b, l, D_MODEL)


# speedup vs baseline: 1.0176x; 1.0176x over previous
import jax
import jax.numpy as jnp
from jax import lax
from jax.experimental import pallas as pl
from jax.experimental.pallas import tpu as pltpu
from jax.experimental.pallas import tpu_sc as plsc

D_MODEL = 768
NUM_INDICES = 4 * 8192

_info = plsc.get_sparse_core_info()
_NC, _NS = _info.num_cores, _info.num_subcores
_NW = _NC * _NS
_PER_W = NUM_INDICES // _NW
_CHUNK = 32
_NCHUNK = _PER_W // _CHUNK
_NBUF = 4


def _gather_body(table_hbm, idx_hbm, out_hbm, idx_v, r0, r1, r2, r3, sem_idx,
                 g0, g1, g2, g3, o0, o1, o2, o3):
    wid = lax.axis_index("s") * _NC + lax.axis_index("c")
    base = wid * _PER_W

    pltpu.async_copy(idx_hbm.at[wid], idx_v, sem_idx).wait()

    bufs = (r0, r1, r2, r3)
    gsems = (g0, g1, g2, g3)
    osems = (o0, o1, o2, o3)

    def gather(j):
        return pltpu.make_async_copy(
            table_hbm.at[idx_v.at[j]], bufs[j % _NBUF], gsems[j % _NBUF])

    def scatter(j):
        return pltpu.make_async_copy(
            bufs[j % _NBUF], out_hbm.at[pl.ds(base + j * _CHUNK, _CHUNK)],
            osems[j % _NBUF])

    gather(0).start()
    gather(1).start()
    for j in range(_NCHUNK):
        gather(j).wait()
        scatter(j).start()
        if j + 2 < _NCHUNK:
            if j >= 2:
                scatter(j - 2).wait()
            gather(j + 2).start()
    for j in range(_NCHUNK - 4, _NCHUNK):
        scatter(j).wait()


def kernel(indices, emb_weight):
    b, l = indices.shape
    idx_flat = indices.reshape(_NW, _NCHUNK, _CHUNK).astype(jnp.int32)

    mesh = plsc.VectorSubcoreMesh(core_axis_name="c", subcore_axis_name="s")
    run = pl.kernel(
        _gather_body,
        mesh=mesh,
        out_type=jax.ShapeDtypeStruct((NUM_INDICES, D_MODEL), jnp.float32),
        scratch_types=(
            [pltpu.VMEM((_NCHUNK, _CHUNK), jnp.int32)]
            + [pltpu.VMEM((_CHUNK, D_MODEL), jnp.float32)] * _NBUF
            + [pltpu.SemaphoreType.DMA] * (1 + 2 * _NBUF)
        ),
    )
    out = run(emb_weight, idx_flat)
    return out.reshape(b, l, D_MODEL)
